# baseline (speedup 1.0000x reference)
import jax
import jax.numpy as jnp
from jax.experimental import pallas as pl

B = 1024
D = 32
U = 100000
RB = 32
NR = B // RB
DA = D + 1


def _sum_body(xa_ref, ka_ref, c_ref, s_ref):
    logits = jnp.dot(xa_ref[...], ka_ref[...],
                     preferred_element_type=jnp.float32)
    e = jnp.exp(logits - c_ref[...])
    s_ref[...] = jnp.sum(e, axis=1, keepdims=True)


def _prob_body(xa_ref, ka_ref, c_ref, r_ref, o_ref):
    logits = jnp.dot(xa_ref[...], ka_ref[...],
                     preferred_element_type=jnp.float32)
    o_ref[...] = jnp.exp(logits - c_ref[...]) * r_ref[...]


def kernel(input_logits, input_targets, kernel, bias):
    x = input_logits.astype(jnp.float32)
    xa = jnp.concatenate([x, jnp.ones((B, 1), jnp.float32)], axis=1)
    ka = jnp.concatenate([kernel.T, bias.astype(jnp.float32)[None, :]],
                         axis=0)
    wmax = jnp.sqrt(jnp.max(jnp.sum(kernel * kernel, axis=1)))
    c = (jnp.sqrt(jnp.sum(x * x, axis=1, keepdims=True)) * wmax
         + jnp.max(bias))

    xa_spec = pl.BlockSpec((RB, DA), lambda i: (i, 0))
    ka_spec = pl.BlockSpec((DA, U), lambda i: (0, 0))
    col_spec = pl.BlockSpec((RB, 1), lambda i: (i, 0))

    s = pl.pallas_call(
        _sum_body,
        grid=(NR,),
        in_specs=[xa_spec, ka_spec, col_spec],
        out_specs=col_spec,
        out_shape=jax.ShapeDtypeStruct((B, 1), jnp.float32),
    )(xa, ka, c)

    probs = pl.pallas_call(
        _prob_body,
        grid=(NR,),
        in_specs=[xa_spec, ka_spec, col_spec, col_spec],
        out_specs=pl.BlockSpec((RB, U), lambda i: (i, 0)),
        out_shape=jax.ShapeDtypeStruct((B, U), jnp.float32),
    )(xa, ka, c, 1.0 / s)
    return probs
